# Initial kernel scaffold; baseline (speedup 1.0000x reference)
import jax
import jax.numpy as jnp
from jax.experimental import pallas as pl


def kernel(coords, ln1_s, ln1_b, Wq, Wk, Wv, Wo, Wg, ln2_s, ln2_b, W1, b1, W2, b2, Wd, bd):
    raise NotImplementedError("write your pallas kernel here")


# speedup vs baseline: 1.7372x; 1.7372x over previous
import jax
import jax.numpy as jnp
from jax.experimental import pallas as pl

N = 4096
K = 12
ED = 128
L = 6
FF = 4 * ED
SIG_LO = 0.05
SIG_HI = 1.0

BN = 256
NB = N // BN
BR = 256
NRB = N // BR


def _dot(a, b, preferred_element_type=None):
    return jax.lax.dot_general(
        a, b, (((1,), (0,)), ((), ())),
        preferred_element_type=preferred_element_type)


def _bf(x):
    return x.astype(jnp.bfloat16).astype(jnp.float32)

def _knn_body(cp_ref, cpt_ref, d2k_ref, idx_ref):
    cb = cp_ref[...]
    ct = cpt_ref[...]
    sqb = jnp.sum(cb * cb, axis=1, keepdims=True)
    sqa = jnp.sum(ct * ct, axis=0, keepdims=True)
    d2 = sqb + sqa - 2.0 * _dot(cb, ct, preferred_element_type=jnp.float32)
    iota = jax.lax.broadcasted_iota(jnp.int32, (BR, N), 1)
    for k in range(K):
        m = jnp.min(d2, axis=1, keepdims=True)
        sel = jnp.where(d2 <= m, iota, N)
        j = jnp.min(sel, axis=1, keepdims=True)
        d2k_ref[:, k:k + 1] = m
        idx_ref[:, k:k + 1] = j
        d2 = jnp.where(iota == j, jnp.float32(jnp.inf), d2)


def _embed_body(d2k_ref, inv_ref, h_ref):
    inv = inv_ref[...]
    for k in range(K):
        c = jnp.maximum(d2k_ref[:, k:k + 1], 0.0)
        h_ref[k] = jnp.exp(-c * inv)


def _ln(x, s, b):
    m = jnp.mean(x, axis=1, keepdims=True)
    xc = x - m
    v = jnp.mean(xc * xc, axis=1, keepdims=True)
    return xc / jnp.sqrt(v + 1e-5) * s + b


def _layer_a_body(h_ref, s_ref, b_ref, wq_ref, wk_ref, wv_ref, wo_ref,
                  wg_ref, h1_ref, nw_ref):
    s_ = s_ref[...]
    b_ = b_ref[...]
    wq = wq_ref[...]
    wk = wk_ref[...]
    wv = wv_ref[...]
    wo = wo_ref[...]
    scale = jnp.float32(ED ** 0.5)
    q = []
    kk = []
    vv = []
    hs = []
    for k in range(K):
        h = h_ref[k]
        hs.append(h)
        x = _ln(h, s_, b_)
        q.append(_bf(_dot(x, wq, preferred_element_type=jnp.float32)))
        kk.append(_bf(_dot(x, wk, preferred_element_type=jnp.float32)))
        vv.append(_bf(_dot(x, wv, preferred_element_type=jnp.float32)))
    nsum = None
    for k in range(K):
        sc = [jnp.sum(q[k] * kk[m], axis=1, keepdims=True) / scale
              for m in range(K)]
        mx = sc[0]
        for m in range(1, K):
            mx = jnp.maximum(mx, sc[m])
        es = [jnp.exp(s0 - mx) for s0 in sc]
        den = es[0]
        for m in range(1, K):
            den = den + es[m]
        o = None
        for m in range(K):
            t = _bf(es[m] / den) * vv[m]
            o = t if o is None else o + t
        h1 = hs[k] + _dot(o, wo, preferred_element_type=jnp.float32)
        h1_ref[k] = h1
        nsum = h1 if nsum is None else nsum + h1
    node = nsum / jnp.float32(K)
    nw_ref[...] = _dot(node, wg_ref[...], preferred_element_type=jnp.float32)


def _layer_b_body(h1_ref, gw_ref, s_ref, b_ref, w1_ref, b1_ref, w2_ref,
                  b2_ref, out_ref):
    s_ = s_ref[...]
    b_ = b_ref[...]
    w1 = w1_ref[...]
    b1 = b1_ref[...]
    w2 = w2_ref[...]
    b2 = b2_ref[...]
    for k in range(K):
        h2 = h1_ref[k] + gw_ref[k]
        y = _ln(h2, s_, b_)
        f = jnp.maximum(_dot(y, w1, preferred_element_type=jnp.float32) + b1, 0.0)
        out_ref[k] = h2 + _dot(f, w2, preferred_element_type=jnp.float32) + b2


def _decode_body(h_ref, wdt_ref, bd_ref, lg_ref):
    wdt = _bf(wdt_ref[...])
    bd = bd_ref[...]
    for k in range(K):
        lg = jnp.sum(_bf(h_ref[k]) * wdt, axis=1, keepdims=True) + bd
        lg_ref[:, k:k + 1] = lg


def _full_spec(shape):
    nd = len(shape)
    return pl.BlockSpec(shape, lambda i, _nd=nd: (0,) * _nd)


def kernel(coords, ln1_s, ln1_b, Wq, Wk, Wv, Wo, Wg, ln2_s, ln2_b, W1, b1,
           W2, b2, Wd, bd):
    f32 = jnp.float32
    cp = jnp.zeros((N, 8), f32).at[:, :3].set(coords)
    cpt = cp.T
    d2k, idx = pl.pallas_call(
        _knn_body,
        grid=(NRB,),
        in_specs=[
            pl.BlockSpec((BR, 8), lambda i: (i, 0)),
            _full_spec((8, N)),
        ],
        out_specs=[
            pl.BlockSpec((BR, K), lambda i: (i, 0)),
            pl.BlockSpec((BR, K), lambda i: (i, 0)),
        ],
        out_shape=[
            jax.ShapeDtypeStruct((N, K), f32),
            jax.ShapeDtypeStruct((N, K), jnp.int32),
        ],
    )(cp, cpt)

    sigmas = jnp.linspace(0.05, 1.0, ED).astype(f32)
    inv2 = (1.0 / (2.0 * sigmas * sigmas)).reshape(1, ED)

    h = pl.pallas_call(
        _embed_body,
        grid=(NB,),
        in_specs=[
            pl.BlockSpec((BN, K), lambda i: (i, 0)),
            _full_spec((1, ED)),
        ],
        out_specs=pl.BlockSpec((K, BN, ED), lambda i: (0, i, 0)),
        out_shape=jax.ShapeDtypeStruct((K, N, ED), f32),
    )(d2k, inv2)

    hspec = pl.BlockSpec((K, BN, ED), lambda i: (0, i, 0))
    layer_a = pl.pallas_call(
        _layer_a_body,
        grid=(NB,),
        in_specs=[
            hspec,
            _full_spec((1, ED)), _full_spec((1, ED)),
            _full_spec((ED, ED)), _full_spec((ED, ED)),
            _full_spec((ED, ED)), _full_spec((ED, ED)),
            _full_spec((ED, ED)),
        ],
        out_specs=[hspec, pl.BlockSpec((BN, ED), lambda i: (i, 0))],
        out_shape=[
            jax.ShapeDtypeStruct((K, N, ED), f32),
            jax.ShapeDtypeStruct((N, ED), f32),
        ],
    )
    layer_b = pl.pallas_call(
        _layer_b_body,
        grid=(NB,),
        in_specs=[
            hspec, hspec,
            _full_spec((1, ED)), _full_spec((1, ED)),
            _full_spec((ED, FF)), _full_spec((1, FF)),
            _full_spec((FF, ED)), _full_spec((1, ED)),
        ],
        out_specs=hspec,
        out_shape=jax.ShapeDtypeStruct((K, N, ED), f32),
    )

    for l in range(L):
        h, nw = layer_a(h, ln1_s[l].reshape(1, ED), ln1_b[l].reshape(1, ED),
                        Wq[l], Wk[l], Wv[l], Wo[l], Wg[l])
        g = jnp.take(nw, idx, axis=0)
        g = jnp.transpose(g, (1, 0, 2))
        h = layer_b(h, g, ln2_s[l].reshape(1, ED), ln2_b[l].reshape(1, ED),
                    W1[l], b1[l].reshape(1, FF), W2[l], b2[l].reshape(1, ED))

    logits = pl.pallas_call(
        _decode_body,
        grid=(NB,),
        in_specs=[hspec, _full_spec((1, ED)), _full_spec((1, 1))],
        out_specs=pl.BlockSpec((BN, K), lambda i: (i, 0)),
        out_shape=jax.ShapeDtypeStruct((N, K), f32),
    )(h, Wd.reshape(1, ED), bd.reshape(1, 1))

    rows = jnp.broadcast_to(jnp.arange(N)[:, None], (N, K))
    dense = jnp.zeros((N, N), f32).at[rows, idx].add(logits)
    dense = dense + dense.T
    return dense


# speedup vs baseline: 2.6337x; 1.5160x over previous
import jax
import jax.numpy as jnp
from jax import lax
from jax.experimental import pallas as pl
from jax.experimental.pallas import tpu as pltpu
from jax.experimental.pallas import tpu_sc as plsc

N = 4096
K = 12
ED = 128
L = 6
FF = 4 * ED
SIG_LO = 0.05
SIG_HI = 1.0

BN = 256
NB = N // BN
BR = 256
NRB = N // BR


def _dot(a, b, preferred_element_type=None):
    return jax.lax.dot_general(
        a, b, (((1,), (0,)), ((), ())),
        preferred_element_type=preferred_element_type)


def _bf(x):
    return x.astype(jnp.bfloat16).astype(jnp.float32)

def _knn_body(cp_ref, cpt_ref, d2k_ref, idx_ref):
    cb = cp_ref[...]
    ct = cpt_ref[...]
    sqb = jnp.sum(cb * cb, axis=1, keepdims=True)
    sqa = jnp.sum(ct * ct, axis=0, keepdims=True)
    d2 = sqb + sqa - 2.0 * _dot(cb, ct, preferred_element_type=jnp.float32)
    iota = jax.lax.broadcasted_iota(jnp.int32, (BR, N), 1)
    for k in range(K):
        m = jnp.min(d2, axis=1, keepdims=True)
        sel = jnp.where(d2 <= m, iota, N)
        j = jnp.min(sel, axis=1, keepdims=True)
        d2k_ref[:, k:k + 1] = m
        idx_ref[:, k:k + 1] = j
        d2 = jnp.where(iota == j, jnp.float32(jnp.inf), d2)


def _embed_body(d2k_ref, inv_ref, h_ref):
    inv = inv_ref[...]
    for k in range(K):
        c = jnp.maximum(d2k_ref[:, k:k + 1], 0.0)
        h_ref[k] = jnp.exp(-c * inv)


def _ln(x, s, b):
    m = jnp.mean(x, axis=1, keepdims=True)
    xc = x - m
    v = jnp.mean(xc * xc, axis=1, keepdims=True)
    return xc / jnp.sqrt(v + 1e-5) * s + b


def _layer_a_body(h_ref, s_ref, b_ref, wq_ref, wk_ref, wv_ref, wo_ref,
                  wg_ref, h1_ref, nw_ref):
    s_ = s_ref[...]
    b_ = b_ref[...]
    wq = wq_ref[...]
    wk = wk_ref[...]
    wv = wv_ref[...]
    wo = wo_ref[...]
    scale = jnp.float32(ED ** 0.5)
    q = []
    kk = []
    vv = []
    hs = []
    for k in range(K):
        h = h_ref[k]
        hs.append(h)
        x = _ln(h, s_, b_)
        q.append(_bf(_dot(x, wq, preferred_element_type=jnp.float32)))
        kk.append(_bf(_dot(x, wk, preferred_element_type=jnp.float32)))
        vv.append(_bf(_dot(x, wv, preferred_element_type=jnp.float32)))
    nsum = None
    for k in range(K):
        sc = [jnp.sum(q[k] * kk[m], axis=1, keepdims=True) / scale
              for m in range(K)]
        mx = sc[0]
        for m in range(1, K):
            mx = jnp.maximum(mx, sc[m])
        es = [jnp.exp(s0 - mx) for s0 in sc]
        den = es[0]
        for m in range(1, K):
            den = den + es[m]
        o = None
        for m in range(K):
            t = _bf(es[m] / den) * vv[m]
            o = t if o is None else o + t
        h1 = hs[k] + _dot(o, wo, preferred_element_type=jnp.float32)
        h1_ref[k] = h1
        nsum = h1 if nsum is None else nsum + h1
    node = nsum / jnp.float32(K)
    nw_ref[...] = _dot(node, wg_ref[...], preferred_element_type=jnp.float32)


def _layer_b_body(h1_ref, gw_ref, s_ref, b_ref, w1_ref, b1_ref, w2_ref,
                  b2_ref, out_ref):
    s_ = s_ref[...]
    b_ = b_ref[...]
    w1 = w1_ref[...]
    b1 = b1_ref[...]
    w2 = w2_ref[...]
    b2 = b2_ref[...]
    for k in range(K):
        h2 = h1_ref[k] + gw_ref[k]
        y = _ln(h2, s_, b_)
        f = jnp.maximum(_dot(y, w1, preferred_element_type=jnp.float32) + b1, 0.0)
        out_ref[k] = h2 + _dot(f, w2, preferred_element_type=jnp.float32) + b2


def _decode_body(h_ref, wdt_ref, bd_ref, lg_ref):
    wdt = _bf(wdt_ref[...])
    bd = bd_ref[...]
    for k in range(K):
        lg = jnp.sum(_bf(h_ref[k]) * wdt, axis=1, keepdims=True) + bd
        lg_ref[:, k:k + 1] = lg


_SC_NC = 2
_SC_NS = 16
_NW = _SC_NC * _SC_NS
_GB = N // _NW


def _gather_body(table_hbm, idxt_hbm, out_hbm, idx_v, rows_v, gsem, ssem0,
                 ssem1):
    wid = lax.axis_index("s") * _SC_NC + lax.axis_index("c")
    base = wid * _GB
    pltpu.sync_copy(idxt_hbm.at[:, pl.ds(base, _GB)], idx_v)
    ssems = [ssem0, ssem1]
    pending = [None, None]
    for k in range(K):
        b = k & 1
        if pending[b] is not None:
            pending[b].wait()
        pltpu.async_copy(table_hbm.at[idx_v.at[k]], rows_v.at[b], gsem).wait()
        cp = pltpu.async_copy(rows_v.at[b], out_hbm.at[k, pl.ds(base, _GB)],
                              ssems[b])
        pending[b] = cp
    pending[0].wait()
    pending[1].wait()


_sc_gather = pl.kernel(
    _gather_body,
    out_type=jax.ShapeDtypeStruct((K, N, ED), jnp.float32),
    mesh=plsc.VectorSubcoreMesh(core_axis_name="c", subcore_axis_name="s"),
    scratch_types=[
        pltpu.VMEM((K, _GB), jnp.int32),
        pltpu.VMEM((2, _GB, ED), jnp.float32),
        pltpu.SemaphoreType.DMA,
        pltpu.SemaphoreType.DMA,
        pltpu.SemaphoreType.DMA,
    ],
)


def _full_spec(shape):
    nd = len(shape)
    return pl.BlockSpec(shape, lambda i, _nd=nd: (0,) * _nd)


def kernel(coords, ln1_s, ln1_b, Wq, Wk, Wv, Wo, Wg, ln2_s, ln2_b, W1, b1,
           W2, b2, Wd, bd):
    f32 = jnp.float32
    cp = jnp.zeros((N, 8), f32).at[:, :3].set(coords)
    cpt = cp.T
    d2k, idx = pl.pallas_call(
        _knn_body,
        grid=(NRB,),
        in_specs=[
            pl.BlockSpec((BR, 8), lambda i: (i, 0)),
            _full_spec((8, N)),
        ],
        out_specs=[
            pl.BlockSpec((BR, K), lambda i: (i, 0)),
            pl.BlockSpec((BR, K), lambda i: (i, 0)),
        ],
        out_shape=[
            jax.ShapeDtypeStruct((N, K), f32),
            jax.ShapeDtypeStruct((N, K), jnp.int32),
        ],
    )(cp, cpt)

    sigmas = jnp.linspace(0.05, 1.0, ED).astype(f32)
    inv2 = (1.0 / (2.0 * sigmas * sigmas)).reshape(1, ED)

    h = pl.pallas_call(
        _embed_body,
        grid=(NB,),
        in_specs=[
            pl.BlockSpec((BN, K), lambda i: (i, 0)),
            _full_spec((1, ED)),
        ],
        out_specs=pl.BlockSpec((K, BN, ED), lambda i: (0, i, 0)),
        out_shape=jax.ShapeDtypeStruct((K, N, ED), f32),
    )(d2k, inv2)

    hspec = pl.BlockSpec((K, BN, ED), lambda i: (0, i, 0))
    layer_a = pl.pallas_call(
        _layer_a_body,
        grid=(NB,),
        in_specs=[
            hspec,
            _full_spec((1, ED)), _full_spec((1, ED)),
            _full_spec((ED, ED)), _full_spec((ED, ED)),
            _full_spec((ED, ED)), _full_spec((ED, ED)),
            _full_spec((ED, ED)),
        ],
        out_specs=[hspec, pl.BlockSpec((BN, ED), lambda i: (i, 0))],
        out_shape=[
            jax.ShapeDtypeStruct((K, N, ED), f32),
            jax.ShapeDtypeStruct((N, ED), f32),
        ],
    )
    layer_b = pl.pallas_call(
        _layer_b_body,
        grid=(NB,),
        in_specs=[
            hspec, hspec,
            _full_spec((1, ED)), _full_spec((1, ED)),
            _full_spec((ED, FF)), _full_spec((1, FF)),
            _full_spec((FF, ED)), _full_spec((1, ED)),
        ],
        out_specs=hspec,
        out_shape=jax.ShapeDtypeStruct((K, N, ED), f32),
    )

    idxt = idx.T
    for l in range(L):
        h, nw = layer_a(h, ln1_s[l].reshape(1, ED), ln1_b[l].reshape(1, ED),
                        Wq[l], Wk[l], Wv[l], Wo[l], Wg[l])
        g = _sc_gather(nw, idxt)
        h = layer_b(h, g, ln2_s[l].reshape(1, ED), ln2_b[l].reshape(1, ED),
                    W1[l], b1[l].reshape(1, FF), W2[l], b2[l].reshape(1, ED))

    logits = pl.pallas_call(
        _decode_body,
        grid=(NB,),
        in_specs=[hspec, _full_spec((1, ED)), _full_spec((1, 1))],
        out_specs=pl.BlockSpec((BN, K), lambda i: (i, 0)),
        out_shape=jax.ShapeDtypeStruct((N, K), f32),
    )(h, Wd.reshape(1, ED), bd.reshape(1, 1))

    rows = jnp.broadcast_to(jnp.arange(N)[:, None], (N, K))
    dense = jnp.zeros((N, N), f32).at[rows, idx].add(logits)
    dense = dense + dense.T
    return dense
